# baseline (speedup 1.0000x reference)
import functools

import jax
import jax.numpy as jnp
from jax import lax
from jax.experimental import pallas as pl
from jax.experimental.pallas import tpu as pltpu
from jax.experimental.pallas import tpu_sc as plsc

B = 1024
D = 16
TILE_I = 128


@functools.cache
def _make_sc_gather():
    info = plsc.get_sparse_core_info()
    nc, ns = info.num_cores, info.num_subcores
    nw = nc * ns
    b_per_w = B // nw

    mesh = plsc.VectorSubcoreMesh(core_axis_name="c", subcore_axis_name="s")
    row_t = jax.ShapeDtypeStruct((B, D), jnp.float32)

    @functools.partial(
        pl.kernel,
        out_type=(row_t, row_t, row_t, row_t),
        mesh=mesh,
        scratch_types=[
            pltpu.VMEM((b_per_w,), jnp.int32),
            pltpu.VMEM((b_per_w,), jnp.int32),
            pltpu.VMEM((b_per_w, D), jnp.float32),
            pltpu.VMEM((b_per_w, D), jnp.float32),
            pltpu.VMEM((b_per_w, D), jnp.float32),
            pltpu.VMEM((b_per_w, D), jnp.float32),
            pltpu.SemaphoreType.DMA,
        ],
        compiler_params=pltpu.CompilerParams(use_tc_tiling_on_sc=False),
    )
    def sc_gather(ent_hbm, ent_type_hbm, ent_emb_hbm, type_emb_hbm,
                  ent_proj_hbm, type_proj_hbm,
                  e_out, t_out, p_out, q_out,
                  idx_e, idx_t, rows_e, rows_t, rows_p, rows_q, sem):
        wid = lax.axis_index("s") * nc + lax.axis_index("c")
        base = wid * b_per_w
        sl = pl.ds(base, b_per_w)
        pltpu.sync_copy(ent_hbm.at[sl], idx_e)
        pltpu.sync_copy(ent_type_hbm.at[sl], idx_t)
        cps = (
            pltpu.async_copy(ent_emb_hbm.at[idx_e], rows_e, sem),
            pltpu.async_copy(type_emb_hbm.at[idx_t], rows_t, sem),
            pltpu.async_copy(ent_proj_hbm.at[idx_e], rows_p, sem),
            pltpu.async_copy(type_proj_hbm.at[idx_t], rows_q, sem),
        )
        for cp in cps:
            cp.wait()
        pltpu.sync_copy(rows_e, e_out.at[sl])
        pltpu.sync_copy(rows_t, t_out.at[sl])
        pltpu.sync_copy(rows_p, p_out.at[sl])
        pltpu.sync_copy(rows_q, q_out.at[sl])

    return sc_gather


def _score_body(e_ref, t_ref, pT_ref, qT_ref, o_ref):
    e = e_ref[...]
    t = t_ref[...]
    pT = pT_ref[...]
    qT = qT_ref[...]
    dn = (((1,), (0,)), ((), ()))
    a = lax.dot_general(e, pT, dn, preferred_element_type=jnp.float32)
    c = lax.dot_general(t, qT, dn, preferred_element_type=jnp.float32)
    u = e - t
    acc = jnp.zeros((TILE_I, B), jnp.float32)
    for d in range(D):
        term = u[:, d:d + 1] + a * pT[d:d + 1, :] - c * qT[d:d + 1, :]
        acc = acc + jnp.abs(term)
    o_ref[...] = acc


def _tc_score(e, t, pT, qT):
    grid = B // TILE_I
    return pl.pallas_call(
        _score_body,
        grid=(grid,),
        in_specs=[
            pl.BlockSpec((TILE_I, D), lambda i: (i, 0)),
            pl.BlockSpec((TILE_I, D), lambda i: (i, 0)),
            pl.BlockSpec((D, B), lambda i: (0, 0)),
            pl.BlockSpec((D, B), lambda i: (0, 0)),
        ],
        out_specs=pl.BlockSpec((TILE_I, B), lambda i: (i, 0)),
        out_shape=jax.ShapeDtypeStruct((B, B), jnp.float32),
    )(e, t, pT, qT)


def kernel(ent, ent_type, ent_emb, type_emb, ent_proj, type_proj):
    ent = ent.astype(jnp.int32)
    ent_type = ent_type.astype(jnp.int32)
    e, t, p, q = _make_sc_gather()(ent, ent_type, ent_emb, type_emb,
                                   ent_proj, type_proj)
    return _tc_score(e, t, p.T, q.T)


# speedup vs baseline: 1.0016x; 1.0016x over previous
import functools

import jax
import jax.numpy as jnp
from jax import lax
from jax.experimental import pallas as pl
from jax.experimental.pallas import tpu as pltpu
from jax.experimental.pallas import tpu_sc as plsc

B = 1024
D = 16
TILE_I = 128


_PACK = 128 // D


@functools.cache
def _make_sc_gather():
    info = plsc.get_sparse_core_info()
    nc, ns = info.num_cores, info.num_subcores
    nw = nc * ns
    b_per_w = B // nw

    mesh = plsc.VectorSubcoreMesh(core_axis_name="c", subcore_axis_name="s")
    row_t = jax.ShapeDtypeStruct((B, D), jnp.float32)

    @functools.partial(
        pl.kernel,
        out_type=(row_t, row_t, row_t, row_t),
        mesh=mesh,
        scratch_types=[
            pltpu.VMEM((b_per_w,), jnp.int32),
            pltpu.VMEM((b_per_w,), jnp.int32),
            pltpu.VMEM((b_per_w, D), jnp.int32),
            pltpu.VMEM((b_per_w, D), jnp.int32),
            pltpu.VMEM((b_per_w, 128), jnp.float32),
            pltpu.VMEM((b_per_w, 128), jnp.float32),
            pltpu.VMEM((b_per_w, 128), jnp.float32),
            pltpu.VMEM((b_per_w, 128), jnp.float32),
            pltpu.VMEM((b_per_w, D), jnp.float32),
            pltpu.VMEM((b_per_w, D), jnp.float32),
            pltpu.VMEM((b_per_w, D), jnp.float32),
            pltpu.VMEM((b_per_w, D), jnp.float32),
            pltpu.SemaphoreType.DMA,
        ],
        compiler_params=pltpu.CompilerParams(needs_layout_passes=False),
    )
    def sc_gather(ebig_hbm, elan_hbm, tbig_hbm, tlan_hbm,
                  ent_emb_hbm, type_emb_hbm, ent_proj_hbm, type_proj_hbm,
                  e_out, t_out, p_out, q_out,
                  ebig_v, tbig_v, elan_v, tlan_v,
                  rows_e, rows_t, rows_p, rows_q,
                  sel_e, sel_t, sel_p, sel_q, sem):
        wid = lax.axis_index("s") * nc + lax.axis_index("c")
        base = wid * b_per_w
        sl = pl.ds(base, b_per_w)
        pltpu.sync_copy(ebig_hbm.at[sl], ebig_v)
        pltpu.sync_copy(elan_hbm.at[sl], elan_v)
        pltpu.sync_copy(tbig_hbm.at[sl], tbig_v)
        pltpu.sync_copy(tlan_hbm.at[sl], tlan_v)
        cps = (
            pltpu.async_copy(ent_emb_hbm.at[ebig_v], rows_e, sem),
            pltpu.async_copy(type_emb_hbm.at[tbig_v], rows_t, sem),
            pltpu.async_copy(ent_proj_hbm.at[ebig_v], rows_p, sem),
            pltpu.async_copy(type_proj_hbm.at[tbig_v], rows_q, sem),
        )
        for cp in cps:
            cp.wait()
        for r in range(b_per_w):
            row_ids = jnp.full((D,), r, jnp.int32)
            el = elan_v[r, :]
            tl = tlan_v[r, :]
            sel_e[r, :] = plsc.load_gather(rows_e, [row_ids, el])
            sel_t[r, :] = plsc.load_gather(rows_t, [row_ids, tl])
            sel_p[r, :] = plsc.load_gather(rows_p, [row_ids, el])
            sel_q[r, :] = plsc.load_gather(rows_q, [row_ids, tl])
        pltpu.sync_copy(sel_e, e_out.at[sl])
        pltpu.sync_copy(sel_t, t_out.at[sl])
        pltpu.sync_copy(sel_p, p_out.at[sl])
        pltpu.sync_copy(sel_q, q_out.at[sl])

    return sc_gather


def _score_body(e_ref, t_ref, pT_ref, qT_ref, o_ref):
    e = e_ref[...]
    t = t_ref[...]
    pT = pT_ref[...]
    qT = qT_ref[...]
    dn = (((1,), (0,)), ((), ()))
    a = lax.dot_general(e, pT, dn, preferred_element_type=jnp.float32)
    c = lax.dot_general(t, qT, dn, preferred_element_type=jnp.float32)
    u = e - t
    acc = jnp.zeros((TILE_I, B), jnp.float32)
    for d in range(D):
        term = u[:, d:d + 1] + a * pT[d:d + 1, :] - c * qT[d:d + 1, :]
        acc = acc + jnp.abs(term)
    o_ref[...] = acc


def _tc_score(e, t, pT, qT):
    grid = B // TILE_I
    return pl.pallas_call(
        _score_body,
        grid=(grid,),
        in_specs=[
            pl.BlockSpec((TILE_I, D), lambda i: (i, 0)),
            pl.BlockSpec((TILE_I, D), lambda i: (i, 0)),
            pl.BlockSpec((D, B), lambda i: (0, 0)),
            pl.BlockSpec((D, B), lambda i: (0, 0)),
        ],
        out_specs=pl.BlockSpec((TILE_I, B), lambda i: (i, 0)),
        out_shape=jax.ShapeDtypeStruct((B, B), jnp.float32),
    )(e, t, pT, qT)


def kernel(ent, ent_type, ent_emb, type_emb, ent_proj, type_proj):
    ent = ent.astype(jnp.int32)
    ent_type = ent_type.astype(jnp.int32)
    lane = jnp.arange(D, dtype=jnp.int32)
    ebig, elan = ent // _PACK, (ent % _PACK)[:, None] * D + lane[None, :]
    tbig, tlan = ent_type // _PACK, (ent_type % _PACK)[:, None] * D + lane[None, :]
    e, t, p, q = _make_sc_gather()(
        ebig, elan, tbig, tlan,
        ent_emb.reshape(-1, 128), type_emb.reshape(-1, 128),
        ent_proj.reshape(-1, 128), type_proj.reshape(-1, 128))
    return _tc_score(e, t, p.T, q.T)


# speedup vs baseline: 1.4865x; 1.4841x over previous
import functools

import jax
import jax.numpy as jnp
from jax import lax
from jax.experimental import pallas as pl
from jax.experimental.pallas import tpu as pltpu
from jax.experimental.pallas import tpu_sc as plsc

B = 1024
D = 16
TILE_I = 128


_PACK = 128 // D


@functools.cache
def _make_sc_gather():
    info = plsc.get_sparse_core_info()
    nc, ns = info.num_cores, info.num_subcores
    nw = nc * ns
    b_per_w = B // nw

    mesh = plsc.VectorSubcoreMesh(core_axis_name="c", subcore_axis_name="s")
    row_t = jax.ShapeDtypeStruct((B, D), jnp.float32)

    @functools.partial(
        pl.kernel,
        out_type=(row_t, row_t, row_t, row_t),
        mesh=mesh,
        scratch_types=[
            pltpu.VMEM((b_per_w,), jnp.int32),
            pltpu.VMEM((b_per_w,), jnp.int32),
            pltpu.VMEM((b_per_w, D), jnp.float32),
            pltpu.VMEM((b_per_w, D), jnp.float32),
            pltpu.VMEM((b_per_w, D), jnp.float32),
            pltpu.VMEM((b_per_w, D), jnp.float32),
            pltpu.SemaphoreType.DMA,
        ],
        compiler_params=pltpu.CompilerParams(needs_layout_passes=False),
    )
    def sc_gather(ent_hbm, ent_type_hbm,
                  ent_emb_hbm, type_emb_hbm, ent_proj_hbm, type_proj_hbm,
                  e_out, t_out, p_out, q_out,
                  eidx_v, tidx_v,
                  sel_e, sel_t, sel_p, sel_q, sem):
        wid = lax.axis_index("s") * nc + lax.axis_index("c")
        base = wid * b_per_w
        sl = pl.ds(base, b_per_w)
        pltpu.sync_copy(ent_hbm.at[sl], eidx_v)
        pltpu.sync_copy(ent_type_hbm.at[sl], tidx_v)
        cps = []
        for r in range(b_per_w):
            chunk = r // 16
            ev = eidx_v[pl.ds(chunk * 16, 16)]
            tv = tidx_v[pl.ds(chunk * 16, 16)]
            es = ev[r % 16]
            ts = tv[r % 16]
            dst = pl.ds(r, 1)
            cps.append(pltpu.async_copy(
                ent_emb_hbm.at[pl.ds(es, 1)], sel_e.at[dst], sem))
            cps.append(pltpu.async_copy(
                type_emb_hbm.at[pl.ds(ts, 1)], sel_t.at[dst], sem))
            cps.append(pltpu.async_copy(
                ent_proj_hbm.at[pl.ds(es, 1)], sel_p.at[dst], sem))
            cps.append(pltpu.async_copy(
                type_proj_hbm.at[pl.ds(ts, 1)], sel_q.at[dst], sem))
        for cp in cps:
            cp.wait()
        pltpu.sync_copy(sel_e, e_out.at[sl])
        pltpu.sync_copy(sel_t, t_out.at[sl])
        pltpu.sync_copy(sel_p, p_out.at[sl])
        pltpu.sync_copy(sel_q, q_out.at[sl])

    return sc_gather


def _score_body(e_ref, t_ref, pT_ref, qT_ref, o_ref):
    e = e_ref[...]
    t = t_ref[...]
    pT = pT_ref[...]
    qT = qT_ref[...]
    dn = (((1,), (0,)), ((), ()))
    a = lax.dot_general(e, pT, dn, preferred_element_type=jnp.float32)
    c = lax.dot_general(t, qT, dn, preferred_element_type=jnp.float32)
    u = e - t
    acc = jnp.zeros((TILE_I, B), jnp.float32)
    for d in range(D):
        term = u[:, d:d + 1] + a * pT[d:d + 1, :] - c * qT[d:d + 1, :]
        acc = acc + jnp.abs(term)
    o_ref[...] = acc


def _tc_score(e, t, pT, qT):
    grid = B // TILE_I
    return pl.pallas_call(
        _score_body,
        grid=(grid,),
        in_specs=[
            pl.BlockSpec((TILE_I, D), lambda i: (i, 0)),
            pl.BlockSpec((TILE_I, D), lambda i: (i, 0)),
            pl.BlockSpec((D, B), lambda i: (0, 0)),
            pl.BlockSpec((D, B), lambda i: (0, 0)),
        ],
        out_specs=pl.BlockSpec((TILE_I, B), lambda i: (i, 0)),
        out_shape=jax.ShapeDtypeStruct((B, B), jnp.float32),
    )(e, t, pT, qT)


def kernel(ent, ent_type, ent_emb, type_emb, ent_proj, type_proj):
    ent = ent.astype(jnp.int32)
    ent_type = ent_type.astype(jnp.int32)
    e, t, p, q = _make_sc_gather()(
        ent, ent_type, ent_emb, type_emb, ent_proj, type_proj)
    return _tc_score(e, t, p.T, q.T)


# speedup vs baseline: 13.1810x; 8.8669x over previous
import functools

import jax
import jax.numpy as jnp
from jax import lax
from jax.experimental import pallas as pl
from jax.experimental.pallas import tpu as pltpu
from jax.experimental.pallas import tpu_sc as plsc

B = 1024
D = 16
TILE_I = 128


_PACK = 128 // D


@functools.cache
def _make_sc_gather():
    info = plsc.get_sparse_core_info()
    nc, ns = info.num_cores, info.num_subcores
    nw = nc * ns
    b_per_w = B // nw

    mesh = plsc.VectorSubcoreMesh(core_axis_name="c", subcore_axis_name="s")
    row_t = jax.ShapeDtypeStruct((B, D), jnp.float32)

    @functools.partial(
        pl.kernel,
        out_type=(row_t, row_t, row_t, row_t),
        mesh=mesh,
        scratch_types=[
            pltpu.VMEM((b_per_w,), jnp.int32),
            pltpu.VMEM((b_per_w,), jnp.int32),
            pltpu.VMEM((b_per_w, D), jnp.float32),
            pltpu.VMEM((b_per_w, D), jnp.float32),
            pltpu.VMEM((b_per_w, D), jnp.float32),
            pltpu.VMEM((b_per_w, D), jnp.float32),
            pltpu.SemaphoreType.DMA,
        ],
        compiler_params=pltpu.CompilerParams(needs_layout_passes=False),
    )
    def sc_gather(ent_hbm, ent_type_hbm,
                  ent_emb_hbm, type_emb_hbm, ent_proj_hbm, type_proj_hbm,
                  e_out, t_out, p_out, q_out,
                  eidx_v, tidx_v,
                  sel_e, sel_t, sel_p, sel_q, sem):
        wid = lax.axis_index("s") * nc + lax.axis_index("c")
        base = wid * b_per_w
        sl = pl.ds(base, b_per_w)
        pltpu.sync_copy(ent_hbm.at[sl], eidx_v)
        pltpu.sync_copy(ent_type_hbm.at[sl], tidx_v)
        cps = []
        for r in range(b_per_w):
            chunk = r // 16
            ev = eidx_v[pl.ds(chunk * 16, 16)]
            tv = tidx_v[pl.ds(chunk * 16, 16)]
            es = ev[r % 16]
            ts = tv[r % 16]
            dst = pl.ds(r, 1)
            cps.append(pltpu.async_copy(
                ent_emb_hbm.at[pl.ds(es, 1)], sel_e.at[dst], sem))
            cps.append(pltpu.async_copy(
                type_emb_hbm.at[pl.ds(ts, 1)], sel_t.at[dst], sem))
            cps.append(pltpu.async_copy(
                ent_proj_hbm.at[pl.ds(es, 1)], sel_p.at[dst], sem))
            cps.append(pltpu.async_copy(
                type_proj_hbm.at[pl.ds(ts, 1)], sel_q.at[dst], sem))
        for cp in cps:
            cp.wait()
        pltpu.sync_copy(sel_e, e_out.at[sl])
        pltpu.sync_copy(sel_t, t_out.at[sl])
        pltpu.sync_copy(sel_p, p_out.at[sl])
        pltpu.sync_copy(sel_q, q_out.at[sl])

    return sc_gather


def _score_body(e_ref, t_ref, pT_ref, qT_ref, o_ref):
    e = e_ref[...]
    t = t_ref[...]
    pT = pT_ref[...]
    qT = qT_ref[...]
    dn = (((1,), (0,)), ((), ()))
    a = lax.dot_general(e, pT, dn, preferred_element_type=jnp.float32)
    c = lax.dot_general(t, qT, dn, preferred_element_type=jnp.float32)
    u = e - t
    acc = jnp.zeros((TILE_I, B), jnp.float32)
    for d in range(D):
        term = u[:, d:d + 1] + a * pT[d:d + 1, :] - c * qT[d:d + 1, :]
        acc = acc + jnp.abs(term)
    o_ref[...] = acc


def _tc_score(e, t, pT, qT):
    grid = B // TILE_I
    return pl.pallas_call(
        _score_body,
        grid=(grid,),
        in_specs=[
            pl.BlockSpec((TILE_I, D), lambda i: (i, 0)),
            pl.BlockSpec((TILE_I, D), lambda i: (i, 0)),
            pl.BlockSpec((D, B), lambda i: (0, 0)),
            pl.BlockSpec((D, B), lambda i: (0, 0)),
        ],
        out_specs=pl.BlockSpec((TILE_I, B), lambda i: (i, 0)),
        out_shape=jax.ShapeDtypeStruct((B, B), jnp.float32),
    )(e, t, pT, qT)


def kernel(ent, ent_type, ent_emb, type_emb, ent_proj, type_proj):
    ent = ent.astype(jnp.int32)
    ent_type = ent_type.astype(jnp.int32)
    e = jnp.take(ent_emb, ent, axis=0)
    t = jnp.take(type_emb, ent_type, axis=0)
    p = jnp.take(ent_proj, ent, axis=0)
    q = jnp.take(type_proj, ent_type, axis=0)
    return _tc_score(e, t, p.T, q.T)
